# baseline (speedup 1.0000x reference)
import functools

import jax
import jax.numpy as jnp
from jax import lax
from jax.experimental import pallas as pl
from jax.experimental.pallas import tpu as pltpu
from jax.experimental.pallas import tpu_sc as plsc

B = 16384
GMF_D = 8
MLP_D = 16
CHUNK = 128
BLK = 2048


def _gather_sc(ug_idx, ig_idx, um_idx, im_idx, gu_t, gi_t, mu_t, mi_t):
    info = plsc.get_sparse_core_info()
    NW = info.num_cores * info.num_subcores
    g_span = (B * GMF_D) // NW
    m_span = (B * MLP_D) // NW
    g_chunks = g_span // CHUNK
    m_chunks = m_span // CHUNK

    mesh = plsc.VectorSubcoreMesh(core_axis_name="c", subcore_axis_name="s")

    @functools.partial(
        pl.kernel,
        mesh=mesh,
        out_type=[
            jax.ShapeDtypeStruct((B * GMF_D,), jnp.float32),
            jax.ShapeDtypeStruct((B * GMF_D,), jnp.float32),
            jax.ShapeDtypeStruct((B * MLP_D,), jnp.float32),
            jax.ShapeDtypeStruct((B * MLP_D,), jnp.float32),
        ],
        scratch_types=[
            pltpu.VMEM((g_span,), jnp.int32),
            pltpu.VMEM((g_span,), jnp.int32),
            pltpu.VMEM((m_span,), jnp.int32),
            pltpu.VMEM((m_span,), jnp.int32),
            pltpu.VMEM((g_span,), jnp.float32),
            pltpu.VMEM((g_span,), jnp.float32),
            pltpu.VMEM((m_span,), jnp.float32),
            pltpu.VMEM((m_span,), jnp.float32),
            pltpu.SemaphoreType.DMA,
        ],
    )
    def gather_kernel(ug_hbm, ig_hbm, um_hbm, im_hbm,
                      gu_tab, gi_tab, mu_tab, mi_tab,
                      gu_out, gi_out, mu_out, mi_out,
                      ug_s, ig_s, um_s, im_s, gu_b, gi_b, mu_b, mi_b, sem):
        wid = lax.axis_index("s") * info.num_cores + lax.axis_index("c")
        g0 = wid * g_span
        m0 = wid * m_span
        pltpu.sync_copy(ug_hbm.at[pl.ds(g0, g_span)], ug_s)
        pltpu.sync_copy(ig_hbm.at[pl.ds(g0, g_span)], ig_s)
        pltpu.sync_copy(um_hbm.at[pl.ds(m0, m_span)], um_s)
        pltpu.sync_copy(im_hbm.at[pl.ds(m0, m_span)], im_s)
        copies = []
        for r in range(g_chunks):
            sl = pl.ds(r * CHUNK, CHUNK)
            copies.append(pltpu.async_copy(gu_tab.at[ug_s.at[sl]], gu_b.at[sl], sem))
            copies.append(pltpu.async_copy(gi_tab.at[ig_s.at[sl]], gi_b.at[sl], sem))
        for r in range(m_chunks):
            sl = pl.ds(r * CHUNK, CHUNK)
            copies.append(pltpu.async_copy(mu_tab.at[um_s.at[sl]], mu_b.at[sl], sem))
            copies.append(pltpu.async_copy(mi_tab.at[im_s.at[sl]], mi_b.at[sl], sem))
        for c in copies:
            c.wait()
        pltpu.sync_copy(gu_b, gu_out.at[pl.ds(g0, g_span)])
        pltpu.sync_copy(gi_b, gi_out.at[pl.ds(g0, g_span)])
        pltpu.sync_copy(mu_b, mu_out.at[pl.ds(m0, m_span)])
        pltpu.sync_copy(mi_b, mi_out.at[pl.ds(m0, m_span)])

    return gather_kernel(ug_idx, ig_idx, um_idx, im_idx, gu_t, gi_t, mu_t, mi_t)


def _head_tc_body(gu, gi, mu, mi, w1u, w1i, b1, w2, b2, wlg, wlh, bl, out):
    gmf = gu[...] * gi[...]
    h = mu[...] @ w1u[...] + mi[...] @ w1i[...] + b1[...]
    h = jnp.maximum(h, 0.0)
    h = h @ w2[...] + b2[...]
    h = jnp.maximum(h, 0.0)
    logits = gmf @ wlg[...] + h @ wlh[...] + bl[...]
    out[...] = jax.nn.sigmoid(logits)


def kernel(user, item, gmf_user_emb, gmf_item_emb, mlp_user_emb, mlp_item_emb,
           W1, b1, W2, b2, Wl, bl):
    u32 = user.astype(jnp.int32)
    i32 = item.astype(jnp.int32)
    ar8 = jnp.arange(GMF_D, dtype=jnp.int32)
    ar16 = jnp.arange(MLP_D, dtype=jnp.int32)
    ug_idx = (u32[:, None] * GMF_D + ar8).reshape(-1)
    ig_idx = (i32[:, None] * GMF_D + ar8).reshape(-1)
    um_idx = (u32[:, None] * MLP_D + ar16).reshape(-1)
    im_idx = (i32[:, None] * MLP_D + ar16).reshape(-1)

    guf, gif, muf, mif = _gather_sc(
        ug_idx, ig_idx, um_idx, im_idx,
        gmf_user_emb.reshape(-1), gmf_item_emb.reshape(-1),
        mlp_user_emb.reshape(-1), mlp_item_emb.reshape(-1))

    gu = guf.reshape(B, GMF_D)
    gi = gif.reshape(B, GMF_D)
    mu = muf.reshape(B, MLP_D)
    mi = mif.reshape(B, MLP_D)

    w1u = W1[:MLP_D]
    w1i = W1[MLP_D:]
    wlg = Wl[:GMF_D]
    wlh = Wl[GMF_D:]
    b1r = b1.reshape(1, -1)
    b2r = b2.reshape(1, -1)
    blr = bl.reshape(1, 1)

    n_blk = B // BLK
    out = pl.pallas_call(
        _head_tc_body,
        grid=(n_blk,),
        in_specs=[
            pl.BlockSpec((BLK, GMF_D), lambda i: (i, 0)),
            pl.BlockSpec((BLK, GMF_D), lambda i: (i, 0)),
            pl.BlockSpec((BLK, MLP_D), lambda i: (i, 0)),
            pl.BlockSpec((BLK, MLP_D), lambda i: (i, 0)),
            pl.BlockSpec((MLP_D, MLP_D), lambda i: (0, 0)),
            pl.BlockSpec((MLP_D, MLP_D), lambda i: (0, 0)),
            pl.BlockSpec((1, MLP_D), lambda i: (0, 0)),
            pl.BlockSpec((MLP_D, GMF_D), lambda i: (0, 0)),
            pl.BlockSpec((1, GMF_D), lambda i: (0, 0)),
            pl.BlockSpec((GMF_D, 1), lambda i: (0, 0)),
            pl.BlockSpec((GMF_D, 1), lambda i: (0, 0)),
            pl.BlockSpec((1, 1), lambda i: (0, 0)),
        ],
        out_specs=pl.BlockSpec((BLK, 1), lambda i: (i, 0)),
        out_shape=jax.ShapeDtypeStruct((B, 1), jnp.float32),
    )(gu, gi, mu, mi, w1u, w1i, b1r, W2, b2r, wlg, wlh, blr)
    return out.reshape(-1)
